# Initial kernel scaffold; baseline (speedup 1.0000x reference)
import jax
import jax.numpy as jnp
from jax.experimental import pallas as pl


def kernel(state, edge_index, W, b, w1, b1, w2, b2, w3, b3):
    raise NotImplementedError("write your pallas kernel here")


# speedup vs baseline: 13.9387x; 13.9387x over previous
import functools

import jax
import jax.numpy as jnp
from jax import lax
from jax.experimental import pallas as pl
from jax.experimental.pallas import tpu as pltpu
from jax.experimental.pallas import tpu_sc as plsc

NN = 10000
EE = 320000
CC = 128
HH = 32
AA = 8

NC = 2
NS = 16
NW = NC * NS
CHUNK = 128
NCH = 79
EPW = NCH * CHUNK
EPAD = NW * EPW
RPT = 640
NPAD = NS * RPT
RB = 1024
GRID = 10

_mesh = plsc.VectorSubcoreMesh(core_axis_name="c", subcore_axis_name="s",
                               num_cores=NC, num_subcores=NS)


def _deg_body(col_hbm, out_hbm, colv, onesv, bounce, deg_sh):
    c = lax.axis_index("c")
    s = lax.axis_index("s")
    w = s * NC + c
    for i in range(CHUNK // 16):
        onesv[pl.ds(i * 16, 16)] = jnp.ones((16,), jnp.float32)
    for i in range(RPT // 16):
        bounce[pl.ds(i * 16, 16)] = jnp.zeros((16,), jnp.float32)
    pltpu.sync_copy(bounce, deg_sh.at[pl.ds(s * RPT, RPT)])
    plsc.subcore_barrier()
    pltpu.sync_copy(col_hbm.at[w], colv)

    def body(j, carry):
        pltpu.sync_copy(onesv, deg_sh.at[colv.at[j]], add=True)
        return carry

    lax.fori_loop(0, NCH, body, 0)
    plsc.subcore_barrier()
    pltpu.sync_copy(deg_sh.at[pl.ds(s * RPT, RPT)], bounce)
    pltpu.sync_copy(bounce, out_hbm.at[pl.ds(c * NPAD + s * RPT, RPT)])


_sc_deg = pl.kernel(
    _deg_body,
    out_type=jax.ShapeDtypeStruct((NC * NPAD,), jnp.float32),
    mesh=_mesh,
    scratch_types=[
        pltpu.VMEM((NCH, CHUNK), jnp.int32),
        pltpu.VMEM((CHUNK,), jnp.float32),
        pltpu.VMEM((RPT,), jnp.float32),
        pltpu.VMEM_SHARED((NPAD,), jnp.float32),
    ],
)


def _seg_body(y_hbm, row_hbm, col_hbm, out_hbm, rowv, colv, gbuf, acc_sh, sem):
    c = lax.axis_index("c")
    s = lax.axis_index("s")
    w = s * NC + c

    def zb(i, carry):
        for t in range(CC // 16):
            gbuf[i, pl.ds(t * 16, 16)] = jnp.zeros((16,), jnp.float32)
        return carry

    lax.fori_loop(0, CHUNK, zb, 0)
    for t in range(RPT // CHUNK):
        pltpu.sync_copy(gbuf, acc_sh.at[pl.ds(s * RPT + t * CHUNK, CHUNK)])
    plsc.subcore_barrier()
    pltpu.sync_copy(row_hbm.at[w], rowv)
    pltpu.sync_copy(col_hbm.at[w], colv)

    def body(j, carry):
        pltpu.async_copy(y_hbm.at[rowv.at[j]], gbuf, sem).wait()
        pltpu.sync_copy(gbuf, acc_sh.at[colv.at[j]], add=True)
        return carry

    lax.fori_loop(0, NCH, body, 0)
    plsc.subcore_barrier()

    def ship(t, carry):
        base = s * RPT + t * CHUNK
        pltpu.sync_copy(acc_sh.at[pl.ds(base, CHUNK)], gbuf)
        pltpu.sync_copy(gbuf, out_hbm.at[pl.ds(c * NPAD + base, CHUNK)])
        return carry

    lax.fori_loop(0, RPT // CHUNK, ship, 0)


_sc_segsum = pl.kernel(
    _seg_body,
    out_type=jax.ShapeDtypeStruct((NC * NPAD, CC), jnp.float32),
    mesh=_mesh,
    scratch_types=[
        pltpu.VMEM((NCH, CHUNK), jnp.int32),
        pltpu.VMEM((NCH, CHUNK), jnp.int32),
        pltpu.VMEM((CHUNK, CC), jnp.float32),
        pltpu.VMEM_SHARED((NPAD, CC), jnp.float32),
        pltpu.SemaphoreType.DMA,
    ],
)


def _xw_body(state_ref, w_ref, deg_ref, y_ref):
    xw = jnp.dot(state_ref[...], w_ref[...], preferred_element_type=jnp.float32)
    dis = lax.rsqrt(deg_ref[0] + deg_ref[1] + 1.0)
    y_ref[...] = xw * dis


def _tc_xw(state, W, deg3):
    return pl.pallas_call(
        _xw_body,
        grid=(GRID,),
        in_specs=[
            pl.BlockSpec((RB, CC), lambda i: (i, 0)),
            pl.BlockSpec((CC, CC), lambda i: (0, 0)),
            pl.BlockSpec((NC, RB, 1), lambda i: (0, i, 0)),
        ],
        out_specs=pl.BlockSpec((RB, CC), lambda i: (i, 0)),
        out_shape=jax.ShapeDtypeStruct((NN, CC), jnp.float32),
    )(state, W, deg3)


def _mlp_body(acc_ref, y_ref, state_ref, deg_ref, b_ref, w1_ref, b1_ref,
              w2_ref, b2_ref, w3_ref, b3_ref, conc_ref):
    dis = lax.rsqrt(deg_ref[0] + deg_ref[1] + 1.0)
    agg = acc_ref[0] + acc_ref[1] + y_ref[...]
    out = jnp.maximum(dis * agg + b_ref[...], 0.0) + state_ref[...]
    h = jnp.dot(out, w1_ref[...], preferred_element_type=jnp.float32) + b1_ref[...]
    h = jax.nn.leaky_relu(h)
    h = jnp.dot(h, w2_ref[...], preferred_element_type=jnp.float32) + b2_ref[...]
    h = jax.nn.leaky_relu(h)
    z = jnp.dot(h, w3_ref[...], preferred_element_type=jnp.float32) + b3_ref[...]
    conc_ref[...] = jax.nn.softplus(z) + 1e-20


def _tc_mlp(acc3, y, state, deg3, b2d, w1, b1d, w2, b2d_, w3, b3d):
    return pl.pallas_call(
        _mlp_body,
        grid=(GRID,),
        in_specs=[
            pl.BlockSpec((NC, RB, CC), lambda i: (0, i, 0)),
            pl.BlockSpec((RB, CC), lambda i: (i, 0)),
            pl.BlockSpec((RB, CC), lambda i: (i, 0)),
            pl.BlockSpec((NC, RB, 1), lambda i: (0, i, 0)),
            pl.BlockSpec((1, CC), lambda i: (0, 0)),
            pl.BlockSpec((CC, HH), lambda i: (0, 0)),
            pl.BlockSpec((1, HH), lambda i: (0, 0)),
            pl.BlockSpec((HH, HH), lambda i: (0, 0)),
            pl.BlockSpec((1, HH), lambda i: (0, 0)),
            pl.BlockSpec((HH, 1), lambda i: (0, 0)),
            pl.BlockSpec((1, 1), lambda i: (0, 0)),
        ],
        out_specs=pl.BlockSpec((RB, 1), lambda i: (i, 0)),
        out_shape=jax.ShapeDtypeStruct((NN, 1), jnp.float32),
    )(acc3, y, state, deg3, b2d, w1, b1d, w2, b2d_, w3, b3d)


def _lanczos_gammaln(x):
    z = x + 2.0
    a = jnp.full_like(z, 0.99999999999980993)
    for k, ck in enumerate((676.5203681218851, -1259.1392167224028,
                            771.32342877765313, -176.61502916214059,
                            12.507343278686905, -0.13857109526572012,
                            9.9843695780195716e-6, 1.5056327351493116e-7)):
        a = a + ck / (z + float(k))
    t = z + 6.5
    res = 0.91893853320467274 + (z - 0.5) * jnp.log(t) - t + jnp.log(a)
    return res - jnp.log(x) - jnp.log(x + 1.0)


def _dir_body(g_ref, conc_ref, act_ref, lp_ref):
    g = g_ref[...]
    conc = conc_ref[...]
    action = g / jnp.sum(g, axis=1, keepdims=True)
    act_ref[...] = action
    lp = jnp.sum((conc - 1.0) * jnp.log(action), axis=1, keepdims=True)
    lp = lp + _lanczos_gammaln(jnp.sum(conc, axis=1, keepdims=True))
    lp = lp - jnp.sum(_lanczos_gammaln(conc), axis=1, keepdims=True)
    lp_ref[...] = lp


def _tc_dirichlet(g, conc2):
    return pl.pallas_call(
        _dir_body,
        out_shape=[
            jax.ShapeDtypeStruct((NN // AA, AA), jnp.float32),
            jax.ShapeDtypeStruct((NN // AA, 1), jnp.float32),
        ],
    )(g, conc2)


def kernel(state, edge_index, W, b, w1, b1, w2, b2, w3, b3):
    row = edge_index[0]
    col = edge_index[1]
    pad = EPAD - EE
    row_p = jnp.concatenate([row, jnp.zeros((pad,), row.dtype)])
    col_p = jnp.concatenate([col, jnp.full((pad,), NN, col.dtype)])
    row_p = row_p.reshape(NW, NCH, CHUNK)
    col_p = col_p.reshape(NW, NCH, CHUNK)

    deg2 = _sc_deg(col_p)
    deg3 = deg2.reshape(NC, NPAD, 1)
    y = _tc_xw(state, W, deg3)
    acc = _sc_segsum(y, row_p, col_p)
    acc3 = acc.reshape(NC, NPAD, CC)
    conc = _tc_mlp(acc3, y, state, deg3, b.reshape(1, CC), w1,
                   b1.reshape(1, HH), w2, b2.reshape(1, HH), w3,
                   b3.reshape(1, 1))
    conc2 = conc.reshape(NN // AA, AA)
    g = jax.random.gamma(jax.random.key(42), conc2)
    action, lp = _tc_dirichlet(g, conc2)
    return action, lp.reshape(NN // AA)


# speedup vs baseline: 15.2766x; 1.0960x over previous
import functools

import jax
import jax.numpy as jnp
from jax import lax
from jax.experimental import pallas as pl
from jax.experimental.pallas import tpu as pltpu
from jax.experimental.pallas import tpu_sc as plsc

NN = 10000
EE = 320000
CC = 128
HH = 32
AA = 8

NC = 2
NS = 16
NW = NC * NS
CHUNK = 128
NCH = 79
EPW = NCH * CHUNK
EPAD = NW * EPW
RPT = 632
NPAD = NS * RPT
RB = 1024
GRID = 10

_mesh = plsc.VectorSubcoreMesh(core_axis_name="c", subcore_axis_name="s",
                               num_cores=NC, num_subcores=NS)


def _deg_body(col_hbm, out_hbm, colv, onesv, bounce, deg_sh):
    c = lax.axis_index("c")
    s = lax.axis_index("s")
    w = s * NC + c
    for i in range(CHUNK // 16):
        onesv[pl.ds(i * 16, 16)] = jnp.ones((16,), jnp.float32)
    for i in range(640 // 16):
        bounce[pl.ds(i * 16, 16)] = jnp.zeros((16,), jnp.float32)
    pltpu.sync_copy(bounce.at[pl.ds(0, RPT)], deg_sh.at[pl.ds(s * RPT, RPT)])
    plsc.subcore_barrier()
    pltpu.sync_copy(col_hbm.at[w], colv)

    def body(j, carry):
        pltpu.sync_copy(onesv, deg_sh.at[colv.at[j]], add=True)
        return carry

    lax.fori_loop(0, NCH, body, 0)
    plsc.subcore_barrier()
    pltpu.sync_copy(deg_sh.at[pl.ds(s * RPT, RPT)], bounce.at[pl.ds(0, RPT)])
    pltpu.sync_copy(bounce.at[pl.ds(0, RPT)],
                    out_hbm.at[pl.ds(c * NPAD + s * RPT, RPT)])


_sc_deg = pl.kernel(
    _deg_body,
    out_type=jax.ShapeDtypeStruct((NC * NPAD,), jnp.float32),
    mesh=_mesh,
    scratch_types=[
        pltpu.VMEM((NCH, CHUNK), jnp.int32),
        pltpu.VMEM((CHUNK,), jnp.float32),
        pltpu.VMEM((640,), jnp.float32),
        pltpu.VMEM_SHARED((NPAD,), jnp.float32),
    ],
)


def _seg_body(y_hbm, row_hbm, col_hbm, out_hbm, ridx0, ridx1, colv,
              gbuf0, gbuf1, acc_sh, sg0, sg1, ss0, ss1, si0, si1):
    c = lax.axis_index("c")
    s = lax.axis_index("s")
    w = s * NC + c

    def zb(i, carry):
        for t in range(CC // 16):
            gbuf0[i, pl.ds(t * 16, 16)] = jnp.zeros((16,), jnp.float32)
        return carry

    lax.fori_loop(0, CHUNK, zb, 0)
    for t in range(4):
        pltpu.sync_copy(gbuf0, acc_sh.at[pl.ds(s * RPT + t * CHUNK, CHUNK)])
    pltpu.sync_copy(gbuf0.at[pl.ds(0, RPT - 4 * CHUNK)],
                    acc_sh.at[pl.ds(s * RPT + 4 * CHUNK, RPT - 4 * CHUNK)])
    plsc.subcore_barrier()
    pltpu.sync_copy(col_hbm.at[w], colv)

    pltpu.sync_copy(row_hbm.at[w].at[0], ridx0)
    pltpu.sync_copy(row_hbm.at[w].at[1], ridx1)
    pltpu.async_copy(y_hbm.at[ridx0], gbuf0, sg0)
    pltpu.async_copy(y_hbm.at[ridx1], gbuf1, sg1)

    def body(k, carry):
        j0 = 2 * k
        j1 = 2 * k + 1
        pltpu.make_async_copy(y_hbm.at[ridx0], gbuf0, sg0).wait()

        @pl.when(j0 + 2 < NCH)
        def _():
            pltpu.async_copy(row_hbm.at[w].at[j0 + 2], ridx0, si0)

        pltpu.sync_copy(gbuf0, acc_sh.at[colv.at[j0]], add=True)

        @pl.when(j0 + 2 < NCH)
        def _():
            pltpu.make_async_copy(row_hbm.at[w].at[j0 + 2], ridx0, si0).wait()
            pltpu.async_copy(y_hbm.at[ridx0], gbuf0, sg0)

        pltpu.make_async_copy(y_hbm.at[ridx1], gbuf1, sg1).wait()

        @pl.when(j1 + 2 < NCH)
        def _():
            pltpu.async_copy(row_hbm.at[w].at[j1 + 2], ridx1, si1)

        pltpu.sync_copy(gbuf1, acc_sh.at[colv.at[j1]], add=True)

        @pl.when(j1 + 2 < NCH)
        def _():
            pltpu.make_async_copy(row_hbm.at[w].at[j1 + 2], ridx1, si1).wait()
            pltpu.async_copy(y_hbm.at[ridx1], gbuf1, sg1)

        return carry

    lax.fori_loop(0, NCH // 2, body, 0)
    pltpu.make_async_copy(y_hbm.at[ridx0], gbuf0, sg0).wait()
    pltpu.sync_copy(gbuf0, acc_sh.at[colv.at[NCH - 1]], add=True)
    plsc.subcore_barrier()

    for t in range(4):
        base = s * RPT + t * CHUNK
        pltpu.sync_copy(acc_sh.at[pl.ds(base, CHUNK)], gbuf0)
        pltpu.sync_copy(gbuf0, out_hbm.at[pl.ds(c * NPAD + base, CHUNK)])
    tail = RPT - 4 * CHUNK
    base = s * RPT + 4 * CHUNK
    pltpu.sync_copy(acc_sh.at[pl.ds(base, tail)], gbuf0.at[pl.ds(0, tail)])
    pltpu.sync_copy(gbuf0.at[pl.ds(0, tail)],
                    out_hbm.at[pl.ds(c * NPAD + base, tail)])


_sc_segsum = pl.kernel(
    _seg_body,
    out_type=jax.ShapeDtypeStruct((NC * NPAD, CC), jnp.float32),
    mesh=_mesh,
    scratch_types=[
        pltpu.VMEM((CHUNK,), jnp.int32),
        pltpu.VMEM((CHUNK,), jnp.int32),
        pltpu.VMEM((NCH, CHUNK), jnp.int32),
        pltpu.VMEM((CHUNK, CC), jnp.float32),
        pltpu.VMEM((CHUNK, CC), jnp.float32),
        pltpu.VMEM_SHARED((NPAD, CC), jnp.float32),
        pltpu.SemaphoreType.DMA,
        pltpu.SemaphoreType.DMA,
        pltpu.SemaphoreType.DMA,
        pltpu.SemaphoreType.DMA,
        pltpu.SemaphoreType.DMA,
        pltpu.SemaphoreType.DMA,
    ],
)


def _xw_body(state_ref, w_ref, deg_ref, y_ref):
    xw = jnp.dot(state_ref[...], w_ref[...], preferred_element_type=jnp.float32)
    dis = lax.rsqrt(deg_ref[0] + deg_ref[1] + 1.0)
    y_ref[...] = xw * dis


def _tc_xw(state, W, deg3):
    return pl.pallas_call(
        _xw_body,
        grid=(GRID,),
        in_specs=[
            pl.BlockSpec((RB, CC), lambda i: (i, 0)),
            pl.BlockSpec((CC, CC), lambda i: (0, 0)),
            pl.BlockSpec((NC, RB, 1), lambda i: (0, i, 0)),
        ],
        out_specs=pl.BlockSpec((RB, CC), lambda i: (i, 0)),
        out_shape=jax.ShapeDtypeStruct((NN, CC), jnp.float32),
    )(state, W, deg3)


def _mlp_body(acc_ref, y_ref, state_ref, deg_ref, b_ref, w1_ref, b1_ref,
              w2_ref, b2_ref, w3_ref, b3_ref, conc_ref):
    dis = lax.rsqrt(deg_ref[0] + deg_ref[1] + 1.0)
    agg = acc_ref[0] + acc_ref[1] + y_ref[...]
    out = jnp.maximum(dis * agg + b_ref[...], 0.0) + state_ref[...]
    h = jnp.dot(out, w1_ref[...], preferred_element_type=jnp.float32) + b1_ref[...]
    h = jax.nn.leaky_relu(h)
    h = jnp.dot(h, w2_ref[...], preferred_element_type=jnp.float32) + b2_ref[...]
    h = jax.nn.leaky_relu(h)
    z = jnp.dot(h, w3_ref[...], preferred_element_type=jnp.float32) + b3_ref[...]
    conc_ref[...] = jax.nn.softplus(z) + 1e-20


def _tc_mlp(acc3, y, state, deg3, b2d, w1, b1d, w2, b2d_, w3, b3d):
    return pl.pallas_call(
        _mlp_body,
        grid=(GRID,),
        in_specs=[
            pl.BlockSpec((NC, RB, CC), lambda i: (0, i, 0)),
            pl.BlockSpec((RB, CC), lambda i: (i, 0)),
            pl.BlockSpec((RB, CC), lambda i: (i, 0)),
            pl.BlockSpec((NC, RB, 1), lambda i: (0, i, 0)),
            pl.BlockSpec((1, CC), lambda i: (0, 0)),
            pl.BlockSpec((CC, HH), lambda i: (0, 0)),
            pl.BlockSpec((1, HH), lambda i: (0, 0)),
            pl.BlockSpec((HH, HH), lambda i: (0, 0)),
            pl.BlockSpec((1, HH), lambda i: (0, 0)),
            pl.BlockSpec((HH, 1), lambda i: (0, 0)),
            pl.BlockSpec((1, 1), lambda i: (0, 0)),
        ],
        out_specs=pl.BlockSpec((RB, 1), lambda i: (i, 0)),
        out_shape=jax.ShapeDtypeStruct((NN, 1), jnp.float32),
    )(acc3, y, state, deg3, b2d, w1, b1d, w2, b2d_, w3, b3d)


def _lanczos_gammaln(x):
    z = x + 2.0
    a = jnp.full_like(z, 0.99999999999980993)
    for k, ck in enumerate((676.5203681218851, -1259.1392167224028,
                            771.32342877765313, -176.61502916214059,
                            12.507343278686905, -0.13857109526572012,
                            9.9843695780195716e-6, 1.5056327351493116e-7)):
        a = a + ck / (z + float(k))
    t = z + 6.5
    res = 0.91893853320467274 + (z - 0.5) * jnp.log(t) - t + jnp.log(a)
    return res - jnp.log(x) - jnp.log(x + 1.0)


def _dir_body(g_ref, conc_ref, act_ref, lp_ref):
    g = g_ref[...]
    conc = conc_ref[...]
    action = g / jnp.sum(g, axis=1, keepdims=True)
    act_ref[...] = action
    lp = jnp.sum((conc - 1.0) * jnp.log(action), axis=1, keepdims=True)
    lp = lp + _lanczos_gammaln(jnp.sum(conc, axis=1, keepdims=True))
    lp = lp - jnp.sum(_lanczos_gammaln(conc), axis=1, keepdims=True)
    lp_ref[...] = lp


def _tc_dirichlet(g, conc2):
    return pl.pallas_call(
        _dir_body,
        out_shape=[
            jax.ShapeDtypeStruct((NN // AA, AA), jnp.float32),
            jax.ShapeDtypeStruct((NN // AA, 1), jnp.float32),
        ],
    )(g, conc2)


def kernel(state, edge_index, W, b, w1, b1, w2, b2, w3, b3):
    row = edge_index[0]
    col = edge_index[1]
    pad = EPAD - EE
    row_p = jnp.concatenate([row, jnp.zeros((pad,), row.dtype)])
    col_p = jnp.concatenate([col, jnp.full((pad,), NN, col.dtype)])
    row_p = row_p.reshape(NW, NCH, CHUNK)
    col_p = col_p.reshape(NW, NCH, CHUNK)

    deg2 = _sc_deg(col_p)
    deg3 = deg2.reshape(NC, NPAD, 1)
    y = _tc_xw(state, W, deg3)
    acc = _sc_segsum(y, row_p, col_p)
    acc3 = acc.reshape(NC, NPAD, CC)
    conc = _tc_mlp(acc3, y, state, deg3, b.reshape(1, CC), w1,
                   b1.reshape(1, HH), w2, b2.reshape(1, HH), w3,
                   b3.reshape(1, 1))
    conc2 = conc.reshape(NN // AA, AA)
    g = jax.random.gamma(jax.random.key(42), conc2)
    action, lp = _tc_dirichlet(g, conc2)
    return action, lp.reshape(NN // AA)
